# baseline (speedup 1.0000x reference)
import jax
import jax.numpy as jnp
from jax import lax
from jax.experimental import pallas as pl
from jax.experimental.pallas import tpu as pltpu
from jax.experimental.pallas import tpu_sc as plsc

_N = 100000
_MAX_DEG = 16
_WALK_LEN = 16
_NUM_CORES = 2
_NUM_SUBCORES = 16
_LANES = 16
_CHUNK = 3200
_NVEC = _CHUNK // _LANES
_NSPLIT = 4
_Q = _CHUNK // _NSPLIT
_NQ = _NVEC // _NSPLIT


def _walk_body(neigh_hbm, deg_hbm, unif_hbm, out_hbm,
               deg_v, cur_v, flat_v, d0_v, u_a, u_b, gath_v,
               sem_deg, sem_ua, sem_ub, sem_g0, sem_g1, sem_g2, sem_g3,
               sem_out):
    wid = lax.axis_index("s") * _NUM_CORES + lax.axis_index("c")
    base = jnp.minimum(wid * _CHUNK, _N - _CHUNK)
    sem_g = (sem_g0, sem_g1, sem_g2, sem_g3)

    cp_deg = pltpu.async_copy(deg_hbm, deg_v, sem_deg)

    def fire_u(t):
        u_ref, u_sem = (u_a, sem_ua) if t % 2 == 0 else (u_b, sem_ub)
        off = pl.multiple_of(t * _N + base, _LANES)
        return pltpu.async_copy(unif_hbm.at[pl.ds(off, _CHUNK)], u_ref, u_sem)

    u_descs = {0: fire_u(0), 1: fire_u(1)}
    cp_deg.wait()

    def fused_pass(t, q, u_ref):
        @plsc.parallel_loop(q * _NQ, (q + 1) * _NQ, unroll=2)
        def _f(j):
            sl = pl.ds(j * _LANES, _LANES)
            if t == 0:
                cur = base + j * _LANES + lax.iota(jnp.int32, _LANES)
            else:
                cur = jnp.where(d0_v[sl] > 0, gath_v[sl], cur_v[sl])
            cur_v[sl] = cur
            if t < _WALK_LEN:
                d0 = plsc.load_gather(deg_v, [cur])
                d = jnp.maximum(d0, 1)
                y = d.astype(jnp.float32) * u_ref[sl]
                i = y.astype(jnp.int32)
                idx = jnp.where(i.astype(jnp.float32) < y, i, i - 1)
                idx = jnp.maximum(jnp.minimum(idx, d - 1), 0)
                flat_v[sl] = cur * _MAX_DEG + idx
                d0_v[sl] = d0

    g_descs = {}
    out_descs = {}
    for t in range(_WALK_LEN + 1):
        u_ref = u_a if t % 2 == 0 else u_b
        if t < _WALK_LEN:
            u_descs[t].wait()
        if t >= 2:
            for q in range(_NSPLIT):
                out_descs[(t - 2, q)].wait()
        for q in range(_NSPLIT):
            qs = pl.ds(q * _Q, _Q)
            if t >= 1:
                g_descs[(t - 1, q)].wait()
            fused_pass(t, q, u_ref)
            if t < _WALK_LEN:
                g_descs[(t, q)] = pltpu.async_copy(
                    neigh_hbm.at[flat_v.at[qs]], gath_v.at[qs], sem_g[q])
            if t >= 1:
                ooff = pl.multiple_of((t - 1) * _N + base + q * _Q, _LANES)
                out_descs[(t - 1, q)] = pltpu.async_copy(
                    cur_v.at[qs], out_hbm.at[pl.ds(ooff, _Q)], sem_out)
        if t + 2 <= _WALK_LEN - 1:
            u_descs[t + 2] = fire_u(t + 2)
    for q in range(_NSPLIT):
        out_descs[(_WALK_LEN - 1, q)].wait()


@jax.jit
def kernel(neighbors, degrees, uniforms):
    mesh = plsc.VectorSubcoreMesh(core_axis_name="c", subcore_axis_name="s")
    walk = pl.kernel(
        _walk_body,
        out_type=jax.ShapeDtypeStruct((_WALK_LEN * _N,), jnp.int32),
        mesh=mesh,
        compiler_params=pltpu.CompilerParams(needs_layout_passes=False),
        scratch_types=[
            pltpu.VMEM((_N,), jnp.int32),
            pltpu.VMEM((_CHUNK,), jnp.int32),
            pltpu.VMEM((_CHUNK,), jnp.int32),
            pltpu.VMEM((_CHUNK,), jnp.int32),
            pltpu.VMEM((_CHUNK,), jnp.float32),
            pltpu.VMEM((_CHUNK,), jnp.float32),
            pltpu.VMEM((_CHUNK,), jnp.int32),
            pltpu.SemaphoreType.DMA,
            pltpu.SemaphoreType.DMA,
            pltpu.SemaphoreType.DMA,
            pltpu.SemaphoreType.DMA,
            pltpu.SemaphoreType.DMA,
            pltpu.SemaphoreType.DMA,
            pltpu.SemaphoreType.DMA,
            pltpu.SemaphoreType.DMA,
        ],
    )
    out = walk(neighbors.reshape(-1), degrees, uniforms.reshape(-1))
    return out.reshape(_WALK_LEN, _N)


# speedup vs baseline: 1.0166x; 1.0166x over previous
import jax
import jax.numpy as jnp
from jax import lax
from jax.experimental import pallas as pl
from jax.experimental.pallas import tpu as pltpu
from jax.experimental.pallas import tpu_sc as plsc

_N = 100000
_MAX_DEG = 16
_WALK_LEN = 16
_NUM_CORES = 2
_NUM_SUBCORES = 16
_LANES = 16
_CHUNK = 3136
_NVEC = _CHUNK // _LANES
_NSPLIT = 2
_Q = _CHUNK // _NSPLIT
_NQ = _NVEC // _NSPLIT


def _walk_body(neigh_hbm, deg_hbm, unif_hbm, out_hbm,
               deg_v, cur_v, flat_v, d0_v, u_a, u_b, gath_v,
               sem_deg, sem_ua, sem_ub, sem_g0, sem_g1, sem_out):
    wid = lax.axis_index("s") * _NUM_CORES + lax.axis_index("c")
    base = jnp.minimum(wid * _CHUNK, _N - _CHUNK)
    sem_g = (sem_g0, sem_g1)

    cp_deg = pltpu.async_copy(deg_hbm, deg_v, sem_deg)

    def fire_u(t):
        u_ref, u_sem = (u_a, sem_ua) if t % 2 == 0 else (u_b, sem_ub)
        off = pl.multiple_of(t * _N + base, _LANES)
        return pltpu.async_copy(unif_hbm.at[pl.ds(off, _CHUNK)], u_ref, u_sem)

    u_descs = {0: fire_u(0), 1: fire_u(1)}
    cp_deg.wait()

    def fused_pass(t, q, u_ref):
        @plsc.parallel_loop(q * _NQ, (q + 1) * _NQ, unroll=2)
        def _f(j):
            sl = pl.ds(j * _LANES, _LANES)
            if t == 0:
                cur = base + j * _LANES + lax.iota(jnp.int32, _LANES)
            else:
                cur = jnp.where(d0_v[sl] > 0, gath_v[sl], cur_v[sl])
            cur_v[sl] = cur
            if t < _WALK_LEN:
                d0 = plsc.load_gather(deg_v, [cur])
                d = jnp.maximum(d0, 1)
                y = d.astype(jnp.float32) * u_ref[sl]
                i = y.astype(jnp.int32)
                idx = jnp.where(i.astype(jnp.float32) < y, i, i - 1)
                idx = jnp.maximum(jnp.minimum(idx, d - 1), 0)
                flat_v[sl] = cur * _MAX_DEG + idx
                d0_v[sl] = d0

    g_descs = {}
    out_descs = {}
    for t in range(_WALK_LEN + 1):
        u_ref = u_a if t % 2 == 0 else u_b
        if t < _WALK_LEN:
            u_descs[t].wait()
        if t >= 2:
            for q in range(_NSPLIT):
                out_descs[(t - 2, q)].wait()
        for q in range(_NSPLIT):
            qs = pl.ds(q * _Q, _Q)
            if t >= 1:
                g_descs[(t - 1, q)].wait()
            fused_pass(t, q, u_ref)
            if t < _WALK_LEN:
                g_descs[(t, q)] = pltpu.async_copy(
                    neigh_hbm.at[flat_v.at[qs]], gath_v.at[qs], sem_g[q])
            if t >= 1:
                ooff = pl.multiple_of((t - 1) * _N + base + q * _Q, _LANES)
                out_descs[(t - 1, q)] = pltpu.async_copy(
                    cur_v.at[qs], out_hbm.at[pl.ds(ooff, _Q)], sem_out)
        if t + 2 <= _WALK_LEN - 1:
            u_descs[t + 2] = fire_u(t + 2)
    for q in range(_NSPLIT):
        out_descs[(_WALK_LEN - 1, q)].wait()


@jax.jit
def kernel(neighbors, degrees, uniforms):
    mesh = plsc.VectorSubcoreMesh(core_axis_name="c", subcore_axis_name="s")
    walk = pl.kernel(
        _walk_body,
        out_type=jax.ShapeDtypeStruct((_WALK_LEN * _N,), jnp.int32),
        mesh=mesh,
        compiler_params=pltpu.CompilerParams(needs_layout_passes=False),
        scratch_types=[
            pltpu.VMEM((_N,), jnp.int32),
            pltpu.VMEM((_CHUNK,), jnp.int32),
            pltpu.VMEM((_CHUNK,), jnp.int32),
            pltpu.VMEM((_CHUNK,), jnp.int32),
            pltpu.VMEM((_CHUNK,), jnp.float32),
            pltpu.VMEM((_CHUNK,), jnp.float32),
            pltpu.VMEM((_CHUNK,), jnp.int32),
            pltpu.SemaphoreType.DMA,
            pltpu.SemaphoreType.DMA,
            pltpu.SemaphoreType.DMA,
            pltpu.SemaphoreType.DMA,
            pltpu.SemaphoreType.DMA,
            pltpu.SemaphoreType.DMA,
        ],
    )
    out = walk(neighbors.reshape(-1), degrees, uniforms.reshape(-1))
    return out.reshape(_WALK_LEN, _N)


# speedup vs baseline: 1.0174x; 1.0007x over previous
import jax
import jax.numpy as jnp
from jax import lax
from jax.experimental import pallas as pl
from jax.experimental.pallas import tpu as pltpu
from jax.experimental.pallas import tpu_sc as plsc

_N = 100000
_MAX_DEG = 16
_WALK_LEN = 16
_NUM_CORES = 2
_NUM_SUBCORES = 16
_LANES = 16
_CHUNK = 3136
_NVEC = _CHUNK // _LANES
_NSPLIT = 2
_Q = _CHUNK // _NSPLIT
_NQ = _NVEC // _NSPLIT


def _walk_body(neigh_hbm, deg_hbm, unif_hbm, out_hbm,
               deg_v, cur_v, flat_v, d0_v, u_a, u_b, gath_v,
               sem_deg, sem_ua, sem_ub, sem_g0, sem_g1, sem_out):
    wid = lax.axis_index("s") * _NUM_CORES + lax.axis_index("c")
    base = jnp.minimum(wid * _CHUNK, _N - _CHUNK)
    sem_g = (sem_g0, sem_g1)

    cp_deg = pltpu.async_copy(deg_hbm, deg_v, sem_deg)

    def fire_u(t):
        u_ref, u_sem = (u_a, sem_ua) if t % 2 == 0 else (u_b, sem_ub)
        off = pl.multiple_of(t * _N + base, _LANES)
        return pltpu.async_copy(unif_hbm.at[pl.ds(off, _CHUNK)], u_ref, u_sem)

    u_descs = {0: fire_u(0), 1: fire_u(1)}
    cp_deg.wait()

    def fused_pass(t, q, u_ref):
        @plsc.parallel_loop(q * _NQ, (q + 1) * _NQ, unroll=2)
        def _f(j):
            sl = pl.ds(j * _LANES, _LANES)
            if t == 0:
                cur = base + j * _LANES + lax.iota(jnp.int32, _LANES)
            else:
                cur = jnp.where(d0_v[sl] > 0, gath_v[sl], cur_v[sl])
            cur_v[sl] = cur
            if t < _WALK_LEN:
                d0 = plsc.load_gather(deg_v, [cur])
                d = jnp.maximum(d0, 1)
                y = d.astype(jnp.float32) * u_ref[sl]
                i = y.astype(jnp.int32)
                idx = jnp.where(i.astype(jnp.float32) < y, i, i - 1)
                idx = jnp.maximum(jnp.minimum(idx, d - 1), 0)
                flat_v[sl] = cur * _MAX_DEG + idx
                d0_v[sl] = d0

    g_descs = {}
    out_descs = {}
    for t in range(_WALK_LEN + 1):
        u_ref = u_a if t % 2 == 0 else u_b
        if t < _WALK_LEN:
            with jax.named_scope("uwait"):
                u_descs[t].wait()
        if t >= 2:
            with jax.named_scope("outdrain"):
                for q in range(_NSPLIT):
                    out_descs[(t - 2, q)].wait()
        for q in range(_NSPLIT):
            qs = pl.ds(q * _Q, _Q)
            if t >= 1:
                with jax.named_scope("gwait"):
                    g_descs[(t - 1, q)].wait()
            with jax.named_scope("fpass"):
                fused_pass(t, q, u_ref)
            if t < _WALK_LEN:
                g_descs[(t, q)] = pltpu.async_copy(
                    neigh_hbm.at[flat_v.at[qs]], gath_v.at[qs], sem_g[q])
            if t >= 1:
                ooff = pl.multiple_of((t - 1) * _N + base + q * _Q, _LANES)
                out_descs[(t - 1, q)] = pltpu.async_copy(
                    cur_v.at[qs], out_hbm.at[pl.ds(ooff, _Q)], sem_out)
        if t + 2 <= _WALK_LEN - 1:
            u_descs[t + 2] = fire_u(t + 2)
    for q in range(_NSPLIT):
        out_descs[(_WALK_LEN - 1, q)].wait()


@jax.jit
def kernel(neighbors, degrees, uniforms):
    mesh = plsc.VectorSubcoreMesh(core_axis_name="c", subcore_axis_name="s")
    walk = pl.kernel(
        _walk_body,
        out_type=jax.ShapeDtypeStruct((_WALK_LEN * _N,), jnp.int32),
        mesh=mesh,
        compiler_params=pltpu.CompilerParams(needs_layout_passes=False),
        scratch_types=[
            pltpu.VMEM((_N,), jnp.int32),
            pltpu.VMEM((_CHUNK,), jnp.int32),
            pltpu.VMEM((_CHUNK,), jnp.int32),
            pltpu.VMEM((_CHUNK,), jnp.int32),
            pltpu.VMEM((_CHUNK,), jnp.float32),
            pltpu.VMEM((_CHUNK,), jnp.float32),
            pltpu.VMEM((_CHUNK,), jnp.int32),
            pltpu.SemaphoreType.DMA,
            pltpu.SemaphoreType.DMA,
            pltpu.SemaphoreType.DMA,
            pltpu.SemaphoreType.DMA,
            pltpu.SemaphoreType.DMA,
            pltpu.SemaphoreType.DMA,
        ],
    )
    out = walk(neighbors.reshape(-1), degrees, uniforms.reshape(-1))
    return out.reshape(_WALK_LEN, _N)
